# baseline (speedup 1.0000x reference)
import functools

import jax
import jax.numpy as jnp
from jax import lax
from jax.experimental import pallas as pl
from jax.experimental.pallas import tpu as pltpu
from jax.experimental.pallas import tpu_sc as plsc

KF = 4
VC = 40


def kernel(logits, noise, x, mask, unmask_idx):
    B, F, V = logits.shape
    L = unmask_idx.shape[1]

    logits_t = jnp.transpose(logits, (1, 2, 0))
    noise_t = jnp.transpose(noise, (1, 2, 0))
    x_t = x.T
    mask_t = mask.T.astype(jnp.int32)
    umi_t = unmask_idx.T

    info = plsc.get_sparse_core_info()
    NC, NS, LN = info.num_cores, info.num_subcores, info.num_lanes
    NW = NC * NS
    assert NW % KF == 0
    WPF = NW // KF
    BW = B // WPF
    NG = BW // LN
    NCH = V // VC
    assert V % VC == 0 and VC % 8 == 0 and BW % 128 == 0

    @functools.partial(
        pl.kernel,
        mesh=plsc.VectorSubcoreMesh(core_axis_name="c", subcore_axis_name="s"),
        compiler_params=pltpu.CompilerParams(
            needs_layout_passes=False, use_tc_tiling_on_sc=True),
        out_type=jax.ShapeDtypeStruct((KF * B,), jnp.int32),
        scratch_types=[
            pltpu.VMEM((VC, BW), jnp.float32),
            pltpu.VMEM((VC, BW), jnp.float32),
            pltpu.VMEM((VC, BW), jnp.float32),
            pltpu.VMEM((VC, BW), jnp.float32),
            pltpu.VMEM((BW,), jnp.int32),
            pltpu.SemaphoreType.DMA,
            pltpu.SemaphoreType.DMA,
        ],
    )
    def sc_sample(logits_hbm, noise_hbm, samples_hbm,
                  lb0, nb0, lb1, nb1, sv, sem_l, sem_n):
        wid = lax.axis_index("s") * NC + lax.axis_index("c")
        f_id = wid // WPF
        bs = (wid % WPF) * BW

        lbufs, nbufs = (lb0, lb1), (nb0, nb1)

        def issue(c):
            buf = c % 2
            return (pltpu.async_copy(
                        logits_hbm.at[f_id, pl.ds(c * VC, VC), pl.ds(bs, BW)],
                        lbufs[buf], sem_l),
                    pltpu.async_copy(
                        noise_hbm.at[f_id, pl.ds(c * VC, VC), pl.ds(bs, BW)],
                        nbufs[buf], sem_n))

        ninf = jnp.full((LN,), -jnp.inf, jnp.float32)
        zero = jnp.zeros((LN,), jnp.int32)
        best = [(ninf, zero)] * NG

        pending = issue(0)
        for c in range(NCH):
            pending[0].wait()
            pending[1].wait()
            buf = c % 2
            if c + 1 < NCH:
                pending = issue(c + 1)
            lbuf, nbuf = lbufs[buf], nbufs[buf]

            def body(vi, carry):
                vglob = c * VC + vi
                out = []
                for g in range(NG):
                    bestv, besti = carry[g]
                    sl = pl.ds(g * LN, LN)
                    v = lbuf[vi, sl] + nbuf[vi, sl]
                    better = v > bestv
                    out.append((jnp.where(better, v, bestv),
                                jnp.where(better, vglob, besti)))
                return tuple(out)

            best = lax.fori_loop(0, VC, body, tuple(best))

        for g in range(NG):
            sv[pl.ds(g * LN, LN)] = best[g][1]
        pltpu.sync_copy(sv, samples_hbm.at[pl.ds(f_id * B + bs, BW)])

    samples_sc = sc_sample(logits_t, noise_t).reshape(KF, B)

    def tc_body(lg_ref, ns_ref, xt_ref, mt_ref, umi_ref,
                newx_ref, newm_ref, fm_ref):
        f = pl.program_id(0) + KF
        val = lg_ref[0] + ns_ref[0]
        maxv = jnp.max(val, axis=0)
        iota_v = lax.broadcasted_iota(jnp.int32, (V, B), 0)
        amax = jnp.min(jnp.where(val == maxv[None, :], iota_v, V), axis=0)
        m = mt_ref[pl.ds(f, 1), :]
        hit = (umi_ref[pl.ds(0, 1), :] == f)
        for l in range(1, L):
            hit = hit | (umi_ref[pl.ds(l, 1), :] == f)
        diff = hit & (m == 0)
        fo = pl.ds(f - KF, 1)
        newx_ref[fo, :] = jnp.where(diff, amax[None, :], xt_ref[pl.ds(f, 1), :])
        newm_ref[fo, :] = jnp.where(hit, 1, m)
        fm_ref[fo, :] = jnp.where(m != 0, 0.0, -jnp.inf)

    FT = F - KF
    new_x_hi, new_mask_hi, fm_hi = pl.pallas_call(
        tc_body,
        grid=(FT,),
        in_specs=[
            pl.BlockSpec((1, V, B), lambda f: (f + KF, 0, 0)),
            pl.BlockSpec((1, V, B), lambda f: (f + KF, 0, 0)),
            pl.BlockSpec((F, B), lambda f: (0, 0)),
            pl.BlockSpec((F, B), lambda f: (0, 0)),
            pl.BlockSpec((L, B), lambda f: (0, 0)),
        ],
        out_specs=[
            pl.BlockSpec((FT, B), lambda f: (0, 0)),
            pl.BlockSpec((FT, B), lambda f: (0, 0)),
            pl.BlockSpec((FT, B), lambda f: (0, 0)),
        ],
        out_shape=[
            jax.ShapeDtypeStruct((FT, B), jnp.int32),
            jax.ShapeDtypeStruct((FT, B), jnp.int32),
            jax.ShapeDtypeStruct((FT, B), jnp.float32),
        ],
        compiler_params=pltpu.CompilerParams(
            dimension_semantics=("arbitrary",)),
    )(logits_t, noise_t, x_t, mask_t, umi_t)

    f_lo = jnp.arange(KF, dtype=jnp.int32)
    hit_lo = (umi_t[None, :, :] == f_lo[:, None, None]).any(axis=1)
    m_lo = mask_t[:KF]
    diff_lo = hit_lo & (m_lo == 0)
    new_x_lo = jnp.where(diff_lo, samples_sc, x_t[:KF])
    new_mask_lo = jnp.where(hit_lo, 1, m_lo)
    fm_lo = jnp.where(m_lo != 0, 0.0, -jnp.inf).astype(jnp.float32)

    new_x_t = jnp.concatenate([new_x_lo, new_x_hi], axis=0)
    new_mask_t = jnp.concatenate([new_mask_lo, new_mask_hi], axis=0)
    fm_t = jnp.concatenate([fm_lo, fm_hi], axis=0)
    return new_x_t.T, new_mask_t.T.astype(bool), fm_t.T


# speedup vs baseline: 1.3207x; 1.3207x over previous
import jax
import jax.numpy as jnp
from jax import lax
from jax.experimental import pallas as pl
from jax.experimental.pallas import tpu as pltpu


def kernel(logits, noise, x, mask, unmask_idx):
    B, F, V = logits.shape
    L = unmask_idx.shape[1]

    logits_t = jnp.transpose(logits, (1, 2, 0))
    noise_t = jnp.transpose(noise, (1, 2, 0))
    x_t = x.T
    mask_t = mask.T.astype(jnp.int32)
    umi_t = unmask_idx.T

    def tc_body(lg_ref, ns_ref, xt_ref, mt_ref, umi_ref,
                newx_ref, newm_ref, fm_ref):
        f = pl.program_id(0)
        val = lg_ref[0] + ns_ref[0]
        maxv = jnp.max(val, axis=0)
        iota_v = lax.broadcasted_iota(jnp.int32, (V, B), 0)
        amax = jnp.min(jnp.where(val == maxv[None, :], iota_v, V), axis=0)
        m = mt_ref[pl.ds(f, 1), :]
        hit = (umi_ref[pl.ds(0, 1), :] == f)
        for l in range(1, L):
            hit = hit | (umi_ref[pl.ds(l, 1), :] == f)
        diff = hit & (m == 0)
        newx_ref[pl.ds(f, 1), :] = jnp.where(
            diff, amax[None, :], xt_ref[pl.ds(f, 1), :])
        newm_ref[pl.ds(f, 1), :] = jnp.where(hit, 1, m)
        fm_ref[pl.ds(f, 1), :] = jnp.where(m != 0, 0.0, -jnp.inf)

    new_x_t, new_mask_t, fm_t = pl.pallas_call(
        tc_body,
        grid=(F,),
        in_specs=[
            pl.BlockSpec((1, V, B), lambda f: (f, 0, 0)),
            pl.BlockSpec((1, V, B), lambda f: (f, 0, 0)),
            pl.BlockSpec((F, B), lambda f: (0, 0)),
            pl.BlockSpec((F, B), lambda f: (0, 0)),
            pl.BlockSpec((L, B), lambda f: (0, 0)),
        ],
        out_specs=[
            pl.BlockSpec((F, B), lambda f: (0, 0)),
            pl.BlockSpec((F, B), lambda f: (0, 0)),
            pl.BlockSpec((F, B), lambda f: (0, 0)),
        ],
        out_shape=[
            jax.ShapeDtypeStruct((F, B), jnp.int32),
            jax.ShapeDtypeStruct((F, B), jnp.int32),
            jax.ShapeDtypeStruct((F, B), jnp.float32),
        ],
        compiler_params=pltpu.CompilerParams(
            dimension_semantics=("arbitrary",)),
    )(logits_t, noise_t, x_t, mask_t, umi_t)
    return new_x_t.T, new_mask_t.T.astype(bool), fm_t.T
